# baseline (speedup 1.0000x reference)
import functools

import jax
import jax.numpy as jnp
from jax import lax
from jax.experimental import pallas as pl
from jax.experimental.pallas import tpu as pltpu
from jax.experimental.pallas import tpu_sc as plsc

N_NODES = 10000
N_EDGES = 320000
HID = 128
N_GRAPHS = 16

NC = 2
NS = 16
NW = NC * NS
RANGE = 313
N_PAD = NW * RANGE
SHIFT = 1 << 14
PADVAL = RANGE * SHIFT

CH_A = 2000
NCH_A = N_EDGES // CH_A
FLUSH = 61440
BUFCAP = 67584
LISTCAP = 322560
CH_B = 128
PAD_B = 3 * CH_B

_SC_PARAMS = pltpu.CompilerParams(needs_layout_passes=False)


def _wid():
    return lax.axis_index("s") * NC + lax.axis_index("c")


def _partition_body(src_hbm, dst_hbm, lists_hbm, counts_hbm, src_v, dst_v, outbuf, cnt_stage):
    wid = _wid()
    lo = wid * RANGE
    hi = lo + RANGE
    lbase = wid * LISTCAP

    def group(j, cnt):
        s = src_v[pl.ds(j * 16, 16)]
        t = dst_v[pl.ds(j * 16, 16)]
        msk = (t >= lo) & (t < hi)
        packed = s + (t - lo) * SHIFT
        cs = plsc.cumsum(msk.astype(jnp.int32))
        plsc.store_scatter(outbuf, [cnt + cs - 1], packed, mask=msk)
        return cnt + cs[15]

    def do_flush(cnt, written):
        pltpu.sync_copy(outbuf.at[pl.ds(0, FLUSH)],
                        lists_hbm.at[pl.ds(pl.multiple_of(lbase + written, 2048), FLUSH)])
        for j in range(128):
            outbuf[pl.ds(j * 16, 16)] = outbuf[pl.ds(FLUSH + j * 16, 16)]
        return cnt - FLUSH, written + FLUSH

    def no_flush(cnt, written):
        return cnt, written

    def chunk(i, carry):
        cnt, written = carry
        off = i * CH_A
        pltpu.sync_copy(src_hbm.at[pl.ds(pl.multiple_of(off, 8), CH_A)], src_v)
        pltpu.sync_copy(dst_hbm.at[pl.ds(pl.multiple_of(off, 8), CH_A)], dst_v)
        cnt = lax.fori_loop(0, CH_A // 16, group, cnt)
        cnt, written = lax.cond(cnt >= FLUSH, do_flush, no_flush, cnt, written)
        return cnt, written

    cnt, written = lax.fori_loop(0, NCH_A, chunk, (jnp.int32(0), jnp.int32(0)))

    padv = jnp.full((16,), PADVAL, jnp.int32)
    for j in range(PAD_B // 16):
        outbuf[pl.ds(cnt + j * 16, 16)] = padv
    cnt_p = ((cnt + PAD_B - 1) // PAD_B) * PAD_B

    def blk(b, _):
        pltpu.sync_copy(outbuf.at[pl.ds(b * 2048, 2048)],
                        lists_hbm.at[pl.ds(pl.multiple_of(lbase + written + b * 2048, 2048), 2048)])
        return 0

    lax.fori_loop(0, (cnt_p + 2047) // 2048, blk, 0)
    total = written + cnt_p
    cnt_stage[...] = jnp.broadcast_to(total, (16,))
    pltpu.sync_copy(cnt_stage, counts_hbm.at[pl.ds(pl.multiple_of(wid * 16, 16), 16)])


def _segmax_body(m_hbm, lists_hbm, counts_hbm, agg_hbm,
                 acc, p0, p1, p2, s0, s1, s2, r0, r1, r2, cnt_stage,
                 g0, g1, g2, l0, l1, l2):
    P = (p0, p1, p2)
    S = (s0, s1, s2)
    RW = (r0, r1, r2)
    G = (g0, g1, g2)
    L = (l0, l1, l2)
    wid = _wid()
    lo = wid * RANGE
    lbase = wid * LISTCAP

    pltpu.sync_copy(counts_hbm.at[pl.ds(pl.multiple_of(wid * 16, 16), 16)], cnt_stage)
    total = cnt_stage[pl.ds(0, 16)][0]
    nch = total >> 7

    ninf = jnp.full((16,), -jnp.inf, jnp.float32)

    def init(i, _):
        acc[pl.ds(i * 16, 16)] = ninf
        return 0

    lax.fori_loop(0, (RANGE + 1) * HID // 16, init, 0)

    def list_load(c, b):
        pltpu.async_copy(
            lists_hbm.at[pl.ds(pl.multiple_of(lbase + c * CH_B, CH_B), CH_B)],
            P[b].at[pl.ds(0, CH_B)], L[b])

    def list_wait(b):
        pltpu.make_async_copy(
            lists_hbm.at[pl.ds(pl.multiple_of(lbase, CH_B), CH_B)],
            P[b].at[pl.ds(0, CH_B)], L[b]).wait()

    def unpack_and_gather(b):
        for j in range(CH_B // 16):
            pv = P[b][pl.ds(j * 16, 16)]
            S[b][pl.ds(j * 16, 16)] = pv & (SHIFT - 1)
        pltpu.async_copy(m_hbm.at[S[b]], RW[b], G[b])

    def gather_wait(b):
        pltpu.make_async_copy(m_hbm.at[S[b]], RW[b], G[b]).wait()

    def accumulate(b):
        rows = RW[b]
        pk_buf = P[b]

        def edge(e, _):
            pk = pk_buf[pl.ds(e, 16)][0]
            base = (pk >> 14) * HID
            for c in range(HID // 16):
                off = base + c * 16
                acc[pl.ds(off, 16)] = jnp.maximum(acc[pl.ds(off, 16)],
                                                  rows[e, pl.ds(c * 16, 16)])
            return 0

        lax.fori_loop(0, CH_B, edge, 0)

    @pl.when(nch > 0)
    def _():
        list_load(0, 0)
        list_load(1, 1)
        list_load(2, 2)
        list_wait(0)
        unpack_and_gather(0)

    def triple(t, _):
        c0 = t * 3
        for k in range(3):
            c = c0 + k
            b = k
            bn = (k + 1) % 3
            gather_wait(b)
            accumulate(b)

            @pl.when(c + 3 < nch)
            def _():
                list_load(c + 3, b)

            @pl.when(c + 1 < nch)
            def _():
                list_wait(bn)
                unpack_and_gather(bn)
        return 0

    lax.fori_loop(0, nch // 3, triple, 0)

    def fix(i, _):
        v = acc[pl.ds(i * 16, 16)]
        acc[pl.ds(i * 16, 16)] = jnp.where(v == -jnp.inf, 0.0, v)
        return 0

    lax.fori_loop(0, RANGE * HID // 16, fix, 0)
    pltpu.sync_copy(acc.at[pl.ds(0, RANGE * HID)],
                    agg_hbm.at[pl.ds(pl.multiple_of(lo * HID, 8), RANGE * HID)])


def _silu(x):
    return x * (1.0 / (1.0 + jnp.exp(-x)))


def _mlp(h, w1, b1, w2, b2):
    t = jnp.maximum(jnp.dot(h, w1, preferred_element_type=jnp.float32) + b1, 0.0)
    return jnp.dot(t, w2, preferred_element_type=jnp.float32) + b2


def _embed_body(x_ref, wemb_ref, bemb_ref, w1_ref, b1_ref, w2_ref, b2_ref,
                h0_ref, m1_ref):
    h0 = _silu(x_ref[...] * wemb_ref[...] + bemb_ref[...])
    h0_ref[...] = h0
    m1_ref[...] = _mlp(h0, w1_ref[...], b1_ref[...], w2_ref[...], b2_ref[...])


def _combine_body(h_ref, agg_ref, w1_ref, b1_ref, w2_ref, b2_ref,
                  h1_ref, m2_ref):
    h1 = h_ref[...] + _silu(agg_ref[...])
    h1_ref[...] = h1
    m2_ref[...] = _mlp(h1, w1_ref[...], b1_ref[...], w2_ref[...], b2_ref[...])


def _final_body(h_ref, agg_ref, batch_ref, out_ref):
    h2 = h_ref[...] + _silu(agg_ref[...])
    gid = lax.broadcasted_iota(jnp.int32, (N_GRAPHS, N_NODES), 0)
    oh = (gid == batch_ref[...]).astype(jnp.float32)
    sums = jnp.dot(oh, h2, preferred_element_type=jnp.float32)
    counts = jnp.sum(oh, axis=1, keepdims=True)
    out_ref[...] = sums / jnp.maximum(counts, 1.0)


_F32 = jnp.float32

_embed = pl.pallas_call(
    _embed_body,
    out_shape=[jax.ShapeDtypeStruct((N_NODES, HID), _F32),
               jax.ShapeDtypeStruct((N_NODES, HID), _F32)],
)

_combine = pl.pallas_call(
    _combine_body,
    out_shape=[jax.ShapeDtypeStruct((N_NODES, HID), _F32),
               jax.ShapeDtypeStruct((N_NODES, HID), _F32)],
)

_final = pl.pallas_call(
    _final_body,
    out_shape=jax.ShapeDtypeStruct((N_GRAPHS, HID), _F32),
)


@functools.lru_cache(maxsize=1)
def _sc_kernels():
    mesh = plsc.VectorSubcoreMesh(core_axis_name="c", subcore_axis_name="s")
    partition = pl.kernel(
        _partition_body,
        out_type=[
            jax.ShapeDtypeStruct((NW * LISTCAP,), jnp.int32),
            jax.ShapeDtypeStruct((NW * 16,), jnp.int32),
        ],
        mesh=mesh,
        compiler_params=_SC_PARAMS,
        scratch_types=[
            pltpu.VMEM((CH_A,), jnp.int32),
            pltpu.VMEM((CH_A,), jnp.int32),
            pltpu.VMEM((BUFCAP,), jnp.int32),
            pltpu.VMEM((16,), jnp.int32),
        ],
    )
    segmax = pl.kernel(
        _segmax_body,
        out_type=jax.ShapeDtypeStruct((N_PAD * HID,), jnp.float32),
        mesh=mesh,
        compiler_params=_SC_PARAMS,
        scratch_types=[
            pltpu.VMEM(((RANGE + 1) * HID,), jnp.float32),
            pltpu.VMEM((CH_B + 16,), jnp.int32),
            pltpu.VMEM((CH_B + 16,), jnp.int32),
            pltpu.VMEM((CH_B + 16,), jnp.int32),
            pltpu.VMEM((CH_B,), jnp.int32),
            pltpu.VMEM((CH_B,), jnp.int32),
            pltpu.VMEM((CH_B,), jnp.int32),
            pltpu.VMEM((CH_B, HID), jnp.float32),
            pltpu.VMEM((CH_B, HID), jnp.float32),
            pltpu.VMEM((CH_B, HID), jnp.float32),
            pltpu.VMEM((16,), jnp.int32),
            pltpu.SemaphoreType.DMA,
            pltpu.SemaphoreType.DMA,
            pltpu.SemaphoreType.DMA,
            pltpu.SemaphoreType.DMA,
            pltpu.SemaphoreType.DMA,
            pltpu.SemaphoreType.DMA,
        ],
    )
    return partition, segmax


def kernel(x, edge_index, batch, W_emb, b_emb,
           c1_W1, c1_b1, c1_W2, c1_b2,
           c2_W1, c2_b1, c2_W2, c2_b2):
    src = edge_index[0].astype(jnp.int32)
    dst = edge_index[1].astype(jnp.int32)
    batch2 = batch.astype(jnp.int32).reshape(1, N_NODES)
    bemb = b_emb.reshape(1, HID)
    c1b1 = c1_b1.reshape(1, HID)
    c1b2 = c1_b2.reshape(1, HID)
    c2b1 = c2_b1.reshape(1, HID)
    c2b2 = c2_b2.reshape(1, HID)

    _partition, _segmax = _sc_kernels()
    lists, counts = _partition(src, dst)
    h0, m1 = _embed(x, W_emb, bemb, c1_W1, c1b1, c1_W2, c1b2)
    agg1 = _segmax(m1, lists, counts).reshape(N_PAD, HID)[:N_NODES]
    h1, m2 = _combine(h0, agg1, c2_W1, c2b1, c2_W2, c2b2)
    agg2 = _segmax(m2, lists, counts).reshape(N_PAD, HID)[:N_NODES]
    return _final(h1, agg2, batch2)


# speedup vs baseline: 1.1668x; 1.1668x over previous
import functools

import jax
import jax.numpy as jnp
from jax import lax
from jax.experimental import pallas as pl
from jax.experimental.pallas import tpu as pltpu
from jax.experimental.pallas import tpu_sc as plsc

N_NODES = 10000
N_EDGES = 320000
HID = 128
N_GRAPHS = 16

NC = 2
NS = 16
NW = NC * NS
RANGE = 313
N_PAD = NW * RANGE
SHIFT = 1 << 14
PADVAL = RANGE * SHIFT

CH_A = 2000
NCH_A = N_EDGES // CH_A
FLUSH = 61440
BUFCAP = 67584
LISTCAP = 322560
CH_B = 256
PAD_B = 2 * CH_B

_SC_PARAMS = pltpu.CompilerParams(needs_layout_passes=False)


def _wid():
    return lax.axis_index("s") * NC + lax.axis_index("c")


def _partition_body(src_hbm, dst_hbm, lists_hbm, counts_hbm, src_v, dst_v, outbuf, cnt_stage):
    wid = _wid()
    lo = wid * RANGE
    hi = lo + RANGE
    lbase = wid * LISTCAP

    def group(j, cnt):
        s = src_v[pl.ds(j * 16, 16)]
        t = dst_v[pl.ds(j * 16, 16)]
        msk = (t >= lo) & (t < hi)
        packed = s + (t - lo) * SHIFT
        cs = plsc.cumsum(msk.astype(jnp.int32))
        plsc.store_scatter(outbuf, [cnt + cs - 1], packed, mask=msk)
        return cnt + cs[15]

    def do_flush(cnt, written):
        pltpu.sync_copy(outbuf.at[pl.ds(0, FLUSH)],
                        lists_hbm.at[pl.ds(pl.multiple_of(lbase + written, 2048), FLUSH)])
        for j in range(128):
            outbuf[pl.ds(j * 16, 16)] = outbuf[pl.ds(FLUSH + j * 16, 16)]
        return cnt - FLUSH, written + FLUSH

    def no_flush(cnt, written):
        return cnt, written

    def chunk(i, carry):
        cnt, written = carry
        off = i * CH_A
        pltpu.sync_copy(src_hbm.at[pl.ds(pl.multiple_of(off, 8), CH_A)], src_v)
        pltpu.sync_copy(dst_hbm.at[pl.ds(pl.multiple_of(off, 8), CH_A)], dst_v)
        cnt = lax.fori_loop(0, CH_A // 16, group, cnt)
        cnt, written = lax.cond(cnt >= FLUSH, do_flush, no_flush, cnt, written)
        return cnt, written

    cnt, written = lax.fori_loop(0, NCH_A, chunk, (jnp.int32(0), jnp.int32(0)))

    padv = jnp.full((16,), PADVAL, jnp.int32)
    for j in range(PAD_B // 16):
        outbuf[pl.ds(cnt + j * 16, 16)] = padv
    cnt_p = ((cnt + PAD_B - 1) // PAD_B) * PAD_B

    def blk(b, _):
        pltpu.sync_copy(outbuf.at[pl.ds(b * 2048, 2048)],
                        lists_hbm.at[pl.ds(pl.multiple_of(lbase + written + b * 2048, 2048), 2048)])
        return 0

    lax.fori_loop(0, (cnt_p + 2047) // 2048, blk, 0)
    total = written + cnt_p
    cnt_stage[...] = jnp.broadcast_to(total, (16,))
    pltpu.sync_copy(cnt_stage, counts_hbm.at[pl.ds(pl.multiple_of(wid * 16, 16), 16)])


def _segmax_body(m_hbm, lists_hbm, counts_hbm, agg_hbm,
                 acc, p0, p1, s0, s1, r0, r1, cnt_stage,
                 g0, g1, l0, l1):
    P = (p0, p1)
    S = (s0, s1)
    RW = (r0, r1)
    G = (g0, g1)
    L = (l0, l1)
    wid = _wid()
    lo = wid * RANGE
    lbase = wid * LISTCAP

    pltpu.sync_copy(counts_hbm.at[pl.ds(pl.multiple_of(wid * 16, 16), 16)], cnt_stage)
    total = cnt_stage[pl.ds(0, 16)][0]
    nch = total // CH_B

    ninf = jnp.full((16,), -jnp.inf, jnp.float32)

    def init(i, _):
        acc[pl.ds(i * 16, 16)] = ninf
        return 0

    lax.fori_loop(0, (RANGE + 1) * HID // 16, init, 0)

    def list_load(c, b):
        pltpu.async_copy(
            lists_hbm.at[pl.ds(pl.multiple_of(lbase + c * CH_B, CH_B), CH_B)],
            P[b].at[pl.ds(0, CH_B)], L[b])

    def list_wait(b):
        pltpu.make_async_copy(
            lists_hbm.at[pl.ds(pl.multiple_of(lbase, CH_B), CH_B)],
            P[b].at[pl.ds(0, CH_B)], L[b]).wait()

    def unpack_and_gather(b):
        for j in range(CH_B // 16):
            pv = P[b][pl.ds(j * 16, 16)]
            S[b][pl.ds(j * 16, 16)] = pv & (SHIFT - 1)
        pltpu.async_copy(m_hbm.at[S[b]], RW[b], G[b])

    def gather_wait(b):
        pltpu.make_async_copy(m_hbm.at[S[b]], RW[b], G[b]).wait()

    def accumulate(b):
        rows = RW[b]
        pk_buf = P[b]

        def grp(g, _):
            pkv = pk_buf[pl.ds(pl.multiple_of(g * 16, 16), 16)]
            dls = (pkv >> 14) * HID
            for k in range(16):
                base = dls[k]
                e = g * 16 + k
                for c in range(HID // 16):
                    off = base + c * 16
                    acc[pl.ds(off, 16)] = jnp.maximum(
                        acc[pl.ds(off, 16)], rows[e, pl.ds(c * 16, 16)])
            return 0

        lax.fori_loop(0, CH_B // 16, grp, 0)

    @pl.when(nch > 0)
    def _():
        list_load(0, 0)
        list_load(1, 1)
        list_wait(0)
        unpack_and_gather(0)

    def pair(t, _):
        for k in range(2):
            c = t * 2 + k
            b = k
            bn = (k + 1) % 2
            gather_wait(b)

            @pl.when(c + 1 < nch)
            def _():
                list_wait(bn)
                unpack_and_gather(bn)

            accumulate(b)

            @pl.when(c + 2 < nch)
            def _():
                list_load(c + 2, b)
        return 0

    lax.fori_loop(0, nch // 2, pair, 0)

    def fix(i, _):
        v = acc[pl.ds(i * 16, 16)]
        acc[pl.ds(i * 16, 16)] = jnp.where(v == -jnp.inf, 0.0, v)
        return 0

    lax.fori_loop(0, RANGE * HID // 16, fix, 0)
    pltpu.sync_copy(acc.at[pl.ds(0, RANGE * HID)],
                    agg_hbm.at[pl.ds(pl.multiple_of(lo * HID, 8), RANGE * HID)])


def _silu(x):
    return x * (1.0 / (1.0 + jnp.exp(-x)))


def _mlp(h, w1, b1, w2, b2):
    t = jnp.maximum(jnp.dot(h, w1, preferred_element_type=jnp.float32) + b1, 0.0)
    return jnp.dot(t, w2, preferred_element_type=jnp.float32) + b2


def _embed_body(x_ref, wemb_ref, bemb_ref, w1_ref, b1_ref, w2_ref, b2_ref,
                h0_ref, m1_ref):
    h0 = _silu(x_ref[...] * wemb_ref[...] + bemb_ref[...])
    h0_ref[...] = h0
    m1_ref[...] = _mlp(h0, w1_ref[...], b1_ref[...], w2_ref[...], b2_ref[...])


def _combine_body(h_ref, agg_ref, w1_ref, b1_ref, w2_ref, b2_ref,
                  h1_ref, m2_ref):
    h1 = h_ref[...] + _silu(agg_ref[...])
    h1_ref[...] = h1
    m2_ref[...] = _mlp(h1, w1_ref[...], b1_ref[...], w2_ref[...], b2_ref[...])


def _final_body(h_ref, agg_ref, batch_ref, out_ref):
    h2 = h_ref[...] + _silu(agg_ref[...])
    gid = lax.broadcasted_iota(jnp.int32, (N_GRAPHS, N_NODES), 0)
    oh = (gid == batch_ref[...]).astype(jnp.float32)
    sums = jnp.dot(oh, h2, preferred_element_type=jnp.float32)
    counts = jnp.sum(oh, axis=1, keepdims=True)
    out_ref[...] = sums / jnp.maximum(counts, 1.0)


_F32 = jnp.float32

_embed = pl.pallas_call(
    _embed_body,
    out_shape=[jax.ShapeDtypeStruct((N_NODES, HID), _F32),
               jax.ShapeDtypeStruct((N_NODES, HID), _F32)],
)

_combine = pl.pallas_call(
    _combine_body,
    out_shape=[jax.ShapeDtypeStruct((N_NODES, HID), _F32),
               jax.ShapeDtypeStruct((N_NODES, HID), _F32)],
)

_final = pl.pallas_call(
    _final_body,
    out_shape=jax.ShapeDtypeStruct((N_GRAPHS, HID), _F32),
)


@functools.lru_cache(maxsize=1)
def _sc_kernels():
    mesh = plsc.VectorSubcoreMesh(core_axis_name="c", subcore_axis_name="s")
    partition = pl.kernel(
        _partition_body,
        out_type=[
            jax.ShapeDtypeStruct((NW * LISTCAP,), jnp.int32),
            jax.ShapeDtypeStruct((NW * 16,), jnp.int32),
        ],
        mesh=mesh,
        compiler_params=_SC_PARAMS,
        scratch_types=[
            pltpu.VMEM((CH_A,), jnp.int32),
            pltpu.VMEM((CH_A,), jnp.int32),
            pltpu.VMEM((BUFCAP,), jnp.int32),
            pltpu.VMEM((16,), jnp.int32),
        ],
    )
    segmax = pl.kernel(
        _segmax_body,
        out_type=jax.ShapeDtypeStruct((N_PAD * HID,), jnp.float32),
        mesh=mesh,
        compiler_params=_SC_PARAMS,
        scratch_types=[
            pltpu.VMEM(((RANGE + 1) * HID,), jnp.float32),
            pltpu.VMEM((CH_B,), jnp.int32),
            pltpu.VMEM((CH_B,), jnp.int32),
            pltpu.VMEM((CH_B,), jnp.int32),
            pltpu.VMEM((CH_B,), jnp.int32),
            pltpu.VMEM((CH_B, HID), jnp.float32),
            pltpu.VMEM((CH_B, HID), jnp.float32),
            pltpu.VMEM((16,), jnp.int32),
            pltpu.SemaphoreType.DMA,
            pltpu.SemaphoreType.DMA,
            pltpu.SemaphoreType.DMA,
            pltpu.SemaphoreType.DMA,
        ],
    )
    return partition, segmax


def kernel(x, edge_index, batch, W_emb, b_emb,
           c1_W1, c1_b1, c1_W2, c1_b2,
           c2_W1, c2_b1, c2_W2, c2_b2):
    src = edge_index[0].astype(jnp.int32)
    dst = edge_index[1].astype(jnp.int32)
    batch2 = batch.astype(jnp.int32).reshape(1, N_NODES)
    bemb = b_emb.reshape(1, HID)
    c1b1 = c1_b1.reshape(1, HID)
    c1b2 = c1_b2.reshape(1, HID)
    c2b1 = c2_b1.reshape(1, HID)
    c2b2 = c2_b2.reshape(1, HID)

    _partition, _segmax = _sc_kernels()
    lists, counts = _partition(src, dst)
    h0, m1 = _embed(x, W_emb, bemb, c1_W1, c1b1, c1_W2, c1b2)
    agg1 = _segmax(m1, lists, counts).reshape(N_PAD, HID)[:N_NODES]
    h1, m2 = _combine(h0, agg1, c2_W1, c2b1, c2_W2, c2b2)
    agg2 = _segmax(m2, lists, counts).reshape(N_PAD, HID)[:N_NODES]
    return _final(h1, agg2, batch2)


# speedup vs baseline: 1.3254x; 1.1359x over previous
import functools

import jax
import jax.numpy as jnp
from jax import lax
from jax.experimental import pallas as pl
from jax.experimental.pallas import tpu as pltpu
from jax.experimental.pallas import tpu_sc as plsc

N_NODES = 10000
N_EDGES = 320000
HID = 128
N_GRAPHS = 16

NC = 2
NS = 16
NW = NC * NS
RANGE = 313
N_PAD = NW * RANGE
SHIFT = 1 << 14
PADVAL = RANGE * SHIFT

CH_A = 2000
NCH_A = N_EDGES // CH_A
FLUSH = 61440
BUFCAP = 67584
LISTCAP = 322560
CH_B = 128
PAD_B = 2 * CH_B

_SC_PARAMS = pltpu.CompilerParams(needs_layout_passes=False)


def _wid():
    return lax.axis_index("s") * NC + lax.axis_index("c")


def _partition_body(src_hbm, dst_hbm, lists_hbm, counts_hbm, src_v, dst_v, outbuf, cnt_stage):
    wid = _wid()
    lo = wid * RANGE
    hi = lo + RANGE
    lbase = wid * LISTCAP

    def group(j, cnt):
        s = src_v[pl.ds(j * 16, 16)]
        t = dst_v[pl.ds(j * 16, 16)]
        msk = (t >= lo) & (t < hi)
        packed = s + (t - lo) * SHIFT
        cs = plsc.cumsum(msk.astype(jnp.int32))
        plsc.store_scatter(outbuf, [cnt + cs - 1], packed, mask=msk)
        return cnt + cs[15]

    def do_flush(cnt, written):
        pltpu.sync_copy(outbuf.at[pl.ds(0, FLUSH)],
                        lists_hbm.at[pl.ds(pl.multiple_of(lbase + written, 2048), FLUSH)])
        for j in range(128):
            outbuf[pl.ds(j * 16, 16)] = outbuf[pl.ds(FLUSH + j * 16, 16)]
        return cnt - FLUSH, written + FLUSH

    def no_flush(cnt, written):
        return cnt, written

    def chunk(i, carry):
        cnt, written = carry
        off = i * CH_A
        pltpu.sync_copy(src_hbm.at[pl.ds(pl.multiple_of(off, 8), CH_A)], src_v)
        pltpu.sync_copy(dst_hbm.at[pl.ds(pl.multiple_of(off, 8), CH_A)], dst_v)
        cnt = lax.fori_loop(0, CH_A // 16, group, cnt)
        cnt, written = lax.cond(cnt >= FLUSH, do_flush, no_flush, cnt, written)
        return cnt, written

    cnt, written = lax.fori_loop(0, NCH_A, chunk, (jnp.int32(0), jnp.int32(0)))

    padv = jnp.full((16,), PADVAL, jnp.int32)
    for j in range(PAD_B // 16):
        outbuf[pl.ds(cnt + j * 16, 16)] = padv
    cnt_p = ((cnt + PAD_B - 1) // PAD_B) * PAD_B

    def blk(b, _):
        pltpu.sync_copy(outbuf.at[pl.ds(b * 2048, 2048)],
                        lists_hbm.at[pl.ds(pl.multiple_of(lbase + written + b * 2048, 2048), 2048)])
        return 0

    lax.fori_loop(0, (cnt_p + 2047) // 2048, blk, 0)
    total = written + cnt_p
    cnt_stage[...] = jnp.broadcast_to(total, (16,))
    pltpu.sync_copy(cnt_stage, counts_hbm.at[pl.ds(pl.multiple_of(wid * 16, 16), 16)])


def _segmax_body(m_hbm, lists_hbm, counts_hbm, agg_hbm,
                 acc, acc2, p0, p1, s0, s1, r0, r1, cnt_stage,
                 g0, g1, l0, l1):
    P = (p0, p1)
    S = (s0, s1)
    RW = (r0, r1)
    G = (g0, g1)
    L = (l0, l1)
    wid = _wid()
    lo = wid * RANGE
    lbase = wid * LISTCAP

    pltpu.sync_copy(counts_hbm.at[pl.ds(pl.multiple_of(wid * 16, 16), 16)], cnt_stage)
    total = cnt_stage[pl.ds(0, 16)][0]
    nch = total // CH_B

    ninf = jnp.full((16,), -jnp.inf, jnp.float32)

    def init(i, _):
        acc[pl.ds(i * 16, 16)] = ninf
        acc2[pl.ds(i * 16, 16)] = ninf
        return 0

    lax.fori_loop(0, (RANGE + 1) * HID // 16, init, 0)

    def list_load(c, b):
        pltpu.async_copy(
            lists_hbm.at[pl.ds(pl.multiple_of(lbase + c * CH_B, CH_B), CH_B)],
            P[b].at[pl.ds(0, CH_B)], L[b])

    def list_wait(b):
        pltpu.make_async_copy(
            lists_hbm.at[pl.ds(pl.multiple_of(lbase, CH_B), CH_B)],
            P[b].at[pl.ds(0, CH_B)], L[b]).wait()

    def unpack_and_gather(b):
        for j in range(CH_B // 16):
            pv = P[b][pl.ds(j * 16, 16)]
            S[b][pl.ds(j * 16, 16)] = pv & (SHIFT - 1)
        pltpu.async_copy(m_hbm.at[S[b]], RW[b], G[b])

    def gather_wait(b):
        pltpu.make_async_copy(m_hbm.at[S[b]], RW[b], G[b]).wait()

    def accumulate(b):
        rows = RW[b]
        pk_buf = P[b]

        def grp(g, _):
            pkv = pk_buf[pl.ds(pl.multiple_of(g * 16, 16), 16)]
            dls = (pkv >> 14) * HID
            for k in range(16):
                base = dls[k]
                e = g * 16 + k
                a = acc if (k & 1) == 0 else acc2
                for c in range(HID // 16):
                    off = base + c * 16
                    a[pl.ds(off, 16)] = jnp.maximum(
                        a[pl.ds(off, 16)], rows[e, pl.ds(c * 16, 16)])
            return 0

        lax.fori_loop(0, CH_B // 16, grp, 0)

    @pl.when(nch > 0)
    def _():
        list_load(0, 0)
        list_load(1, 1)
        list_wait(0)
        unpack_and_gather(0)

    def pair(t, _):
        for k in range(2):
            c = t * 2 + k
            b = k
            bn = (k + 1) % 2
            gather_wait(b)

            @pl.when(c + 1 < nch)
            def _():
                list_wait(bn)
                unpack_and_gather(bn)

            accumulate(b)

            @pl.when(c + 2 < nch)
            def _():
                list_load(c + 2, b)
        return 0

    lax.fori_loop(0, nch // 2, pair, 0)

    def fix(i, _):
        v = jnp.maximum(acc[pl.ds(i * 16, 16)], acc2[pl.ds(i * 16, 16)])
        acc[pl.ds(i * 16, 16)] = jnp.where(v == -jnp.inf, 0.0, v)
        return 0

    lax.fori_loop(0, RANGE * HID // 16, fix, 0)
    pltpu.sync_copy(acc.at[pl.ds(0, RANGE * HID)],
                    agg_hbm.at[pl.ds(pl.multiple_of(lo * HID, 8), RANGE * HID)])


def _silu(x):
    return x * (1.0 / (1.0 + jnp.exp(-x)))


def _mlp(h, w1, b1, w2, b2):
    t = jnp.maximum(jnp.dot(h, w1, preferred_element_type=jnp.float32) + b1, 0.0)
    return jnp.dot(t, w2, preferred_element_type=jnp.float32) + b2


def _embed_body(x_ref, wemb_ref, bemb_ref, w1_ref, b1_ref, w2_ref, b2_ref,
                h0_ref, m1_ref):
    h0 = _silu(x_ref[...] * wemb_ref[...] + bemb_ref[...])
    h0_ref[...] = h0
    m1_ref[...] = _mlp(h0, w1_ref[...], b1_ref[...], w2_ref[...], b2_ref[...])


def _combine_body(h_ref, agg_ref, w1_ref, b1_ref, w2_ref, b2_ref,
                  h1_ref, m2_ref):
    h1 = h_ref[...] + _silu(agg_ref[...])
    h1_ref[...] = h1
    m2_ref[...] = _mlp(h1, w1_ref[...], b1_ref[...], w2_ref[...], b2_ref[...])


def _final_body(h_ref, agg_ref, batch_ref, out_ref):
    h2 = h_ref[...] + _silu(agg_ref[...])
    gid = lax.broadcasted_iota(jnp.int32, (N_GRAPHS, N_NODES), 0)
    oh = (gid == batch_ref[...]).astype(jnp.float32)
    sums = jnp.dot(oh, h2, preferred_element_type=jnp.float32)
    counts = jnp.sum(oh, axis=1, keepdims=True)
    out_ref[...] = sums / jnp.maximum(counts, 1.0)


_F32 = jnp.float32

_embed = pl.pallas_call(
    _embed_body,
    out_shape=[jax.ShapeDtypeStruct((N_NODES, HID), _F32),
               jax.ShapeDtypeStruct((N_NODES, HID), _F32)],
)

_combine = pl.pallas_call(
    _combine_body,
    out_shape=[jax.ShapeDtypeStruct((N_NODES, HID), _F32),
               jax.ShapeDtypeStruct((N_NODES, HID), _F32)],
)

_final = pl.pallas_call(
    _final_body,
    out_shape=jax.ShapeDtypeStruct((N_GRAPHS, HID), _F32),
)


@functools.lru_cache(maxsize=1)
def _sc_kernels():
    mesh = plsc.VectorSubcoreMesh(core_axis_name="c", subcore_axis_name="s")
    partition = pl.kernel(
        _partition_body,
        out_type=[
            jax.ShapeDtypeStruct((NW * LISTCAP,), jnp.int32),
            jax.ShapeDtypeStruct((NW * 16,), jnp.int32),
        ],
        mesh=mesh,
        compiler_params=_SC_PARAMS,
        scratch_types=[
            pltpu.VMEM((CH_A,), jnp.int32),
            pltpu.VMEM((CH_A,), jnp.int32),
            pltpu.VMEM((BUFCAP,), jnp.int32),
            pltpu.VMEM((16,), jnp.int32),
        ],
    )
    segmax = pl.kernel(
        _segmax_body,
        out_type=jax.ShapeDtypeStruct((N_PAD * HID,), jnp.float32),
        mesh=mesh,
        compiler_params=_SC_PARAMS,
        scratch_types=[
            pltpu.VMEM(((RANGE + 1) * HID,), jnp.float32),
            pltpu.VMEM(((RANGE + 1) * HID,), jnp.float32),
            pltpu.VMEM((CH_B,), jnp.int32),
            pltpu.VMEM((CH_B,), jnp.int32),
            pltpu.VMEM((CH_B,), jnp.int32),
            pltpu.VMEM((CH_B,), jnp.int32),
            pltpu.VMEM((CH_B, HID), jnp.float32),
            pltpu.VMEM((CH_B, HID), jnp.float32),
            pltpu.VMEM((16,), jnp.int32),
            pltpu.SemaphoreType.DMA,
            pltpu.SemaphoreType.DMA,
            pltpu.SemaphoreType.DMA,
            pltpu.SemaphoreType.DMA,
        ],
    )
    return partition, segmax


def kernel(x, edge_index, batch, W_emb, b_emb,
           c1_W1, c1_b1, c1_W2, c1_b2,
           c2_W1, c2_b1, c2_W2, c2_b2):
    src = edge_index[0].astype(jnp.int32)
    dst = edge_index[1].astype(jnp.int32)
    batch2 = batch.astype(jnp.int32).reshape(1, N_NODES)
    bemb = b_emb.reshape(1, HID)
    c1b1 = c1_b1.reshape(1, HID)
    c1b2 = c1_b2.reshape(1, HID)
    c2b1 = c2_b1.reshape(1, HID)
    c2b2 = c2_b2.reshape(1, HID)

    _partition, _segmax = _sc_kernels()
    lists, counts = _partition(src, dst)
    h0, m1 = _embed(x, W_emb, bemb, c1_W1, c1b1, c1_W2, c1b2)
    agg1 = _segmax(m1, lists, counts).reshape(N_PAD, HID)[:N_NODES]
    h1, m2 = _combine(h0, agg1, c2_W1, c2b1, c2_W2, c2b2)
    agg2 = _segmax(m2, lists, counts).reshape(N_PAD, HID)[:N_NODES]
    return _final(h1, agg2, batch2)


# speedup vs baseline: 1.3928x; 1.0508x over previous
import functools

import jax
import jax.numpy as jnp
from jax import lax
from jax.experimental import pallas as pl
from jax.experimental.pallas import tpu as pltpu
from jax.experimental.pallas import tpu_sc as plsc

N_NODES = 10000
N_EDGES = 320000
HID = 128
N_GRAPHS = 16

NC = 2
NS = 16
NW = NC * NS
RANGE = 313
N_PAD = NW * RANGE
SHIFT = 1 << 14
PADVAL = RANGE * SHIFT

CH_A = 2000
NCH_A = N_EDGES // CH_A
SEG = 4096
BUF2 = 16 * SEG
TH = SEG - 126
OVFCAP = 82 * BUF2
MRGCAP = 67584
LISTCAP = OVFCAP + MRGCAP
BUFA = BUF2 + 2304
CH_B = 128
PAD_B = 2 * CH_B

_SC_PARAMS = pltpu.CompilerParams(needs_layout_passes=False)


def _wid():
    return lax.axis_index("s") * NC + lax.axis_index("c")


def _partition_body(src_hbm, dst_hbm, lists_hbm, counts_hbm, src_v, dst_v, outbuf, cnt_stage):
    wid = _wid()
    lo = wid * RANGE
    hi = lo + RANGE
    lbase = wid * LISTCAP
    lanes = lax.iota(jnp.int32, 16)
    lane_base = lanes * SEG
    padv = jnp.full((16,), PADVAL, jnp.int32)

    def prefill(i, _):
        outbuf[pl.ds(i * 16, 16)] = padv
        return 0

    lax.fori_loop(0, BUF2 // 16, prefill, 0)

    def group(j, lc):
        s = src_v[pl.ds(j * 16, 16)]
        t = dst_v[pl.ds(j * 16, 16)]
        msk = (t >= lo) & (t < hi)
        packed = s + (t - lo) * SHIFT
        plsc.store_scatter(outbuf, [lane_base + lc], packed, mask=msk)
        return lc + msk.astype(jnp.int32)

    def do_flush(lc, ovf_w):
        pltpu.sync_copy(outbuf.at[pl.ds(0, BUF2)],
                        lists_hbm.at[pl.ds(pl.multiple_of(lbase + ovf_w, 128), BUF2)])
        lax.fori_loop(0, BUF2 // 16, prefill, 0)
        return jnp.zeros((16,), jnp.int32), ovf_w + BUF2

    def no_flush(lc, ovf_w):
        return lc, ovf_w

    def chunk(i, carry):
        lc, ovf_w = carry
        off = i * CH_A
        pltpu.sync_copy(src_hbm.at[pl.ds(pl.multiple_of(off, 8), CH_A)], src_v)
        pltpu.sync_copy(dst_hbm.at[pl.ds(pl.multiple_of(off, 8), CH_A)], dst_v)
        lc = lax.fori_loop(0, CH_A // 16, group, lc)
        big = (lc >= TH).astype(jnp.int32)
        anyb = big[0]
        for k in range(1, 16):
            anyb = anyb | big[k]
        lc, ovf_w = lax.cond(anyb > 0, do_flush, no_flush, lc, ovf_w)
        return lc, ovf_w

    lc, ovf_w = lax.fori_loop(0, NCH_A, chunk,
                              (jnp.zeros((16,), jnp.int32), jnp.int32(0)))

    tot = jnp.int32(0)
    for L in range(16):
        cl = lc[L]
        src_base = L * SEG
        dst_base = tot

        def cp(v, _):
            vec = outbuf[pl.ds(src_base + v * 16, 16)]
            outbuf[pl.ds(dst_base + v * 16, 16)] = vec
            return 0

        lax.fori_loop(0, (cl + 15) >> 4, cp, 0)
        tot = tot + cl

    for j in range(PAD_B // 16):
        outbuf[pl.ds(tot + j * 16, 16)] = padv
    cnt_p = ((tot + PAD_B - 1) >> 8) << 8

    def blk(b, _):
        pltpu.sync_copy(
            outbuf.at[pl.ds(b * 2048, 2048)],
            lists_hbm.at[pl.ds(pl.multiple_of(lbase + OVFCAP + b * 2048, 2048), 2048)])
        return 0

    lax.fori_loop(0, (cnt_p + 2047) >> 11, blk, 0)
    cnt_stage[...] = jnp.where(lanes == 0, ovf_w,
                               jnp.where(lanes == 1, cnt_p, 0))
    pltpu.sync_copy(cnt_stage, counts_hbm.at[pl.ds(pl.multiple_of(wid * 16, 16), 16)])


def _segmax_body(m_hbm, lists_hbm, counts_hbm, agg_hbm,
                 acc, acc2, p0, p1, s0, s1, r0, r1, cnt_stage,
                 g0, g1, l0, l1):
    P = (p0, p1)
    S = (s0, s1)
    RW = (r0, r1)
    G = (g0, g1)
    L = (l0, l1)
    wid = _wid()
    lo = wid * RANGE
    lbase = wid * LISTCAP

    pltpu.sync_copy(counts_hbm.at[pl.ds(pl.multiple_of(wid * 16, 16), 16)], cnt_stage)
    cv = cnt_stage[pl.ds(0, 16)]
    novf = cv[0] >> 7
    nch = novf + (cv[1] >> 7)

    ninf = jnp.full((16,), -jnp.inf, jnp.float32)

    def init(i, _):
        acc[pl.ds(i * 16, 16)] = ninf
        acc2[pl.ds(i * 16, 16)] = ninf
        return 0

    lax.fori_loop(0, (RANGE + 1) * HID // 16, init, 0)

    def chunk_addr(c):
        return lbase + jnp.where(c < novf, c * CH_B,
                                 OVFCAP + (c - novf) * CH_B)

    def list_load(c, b):
        pltpu.async_copy(
            lists_hbm.at[pl.ds(pl.multiple_of(chunk_addr(c), CH_B), CH_B)],
            P[b].at[pl.ds(0, CH_B)], L[b])

    def list_wait(b):
        pltpu.make_async_copy(
            lists_hbm.at[pl.ds(pl.multiple_of(lbase, CH_B), CH_B)],
            P[b].at[pl.ds(0, CH_B)], L[b]).wait()

    def unpack_and_gather(b):
        for j in range(CH_B // 16):
            pv = P[b][pl.ds(j * 16, 16)]
            S[b][pl.ds(j * 16, 16)] = pv & (SHIFT - 1)
        pltpu.async_copy(m_hbm.at[S[b]], RW[b], G[b])

    def gather_wait(b):
        pltpu.make_async_copy(m_hbm.at[S[b]], RW[b], G[b]).wait()

    def accumulate(b):
        rows = RW[b]
        pk_buf = P[b]

        def grp(g, _):
            pkv = pk_buf[pl.ds(pl.multiple_of(g * 16, 16), 16)]
            dls = (pkv >> 14) * HID
            for k in range(16):
                base = dls[k]
                e = g * 16 + k
                a = acc if (k & 1) == 0 else acc2
                for c in range(HID // 16):
                    off = base + c * 16
                    a[pl.ds(off, 16)] = jnp.maximum(
                        a[pl.ds(off, 16)], rows[e, pl.ds(c * 16, 16)])
            return 0

        lax.fori_loop(0, CH_B // 16, grp, 0)

    @pl.when(nch > 0)
    def _():
        list_load(0, 0)
        list_load(1, 1)
        list_wait(0)
        unpack_and_gather(0)

    def pair(t, _):
        for k in range(2):
            c = t * 2 + k
            b = k
            bn = (k + 1) % 2
            gather_wait(b)

            @pl.when(c + 1 < nch)
            def _():
                list_wait(bn)
                unpack_and_gather(bn)

            accumulate(b)

            @pl.when(c + 2 < nch)
            def _():
                list_load(c + 2, b)
        return 0

    lax.fori_loop(0, nch // 2, pair, 0)

    def fix(i, _):
        v = jnp.maximum(acc[pl.ds(i * 16, 16)], acc2[pl.ds(i * 16, 16)])
        acc[pl.ds(i * 16, 16)] = jnp.where(v == -jnp.inf, 0.0, v)
        return 0

    lax.fori_loop(0, RANGE * HID // 16, fix, 0)
    pltpu.sync_copy(acc.at[pl.ds(0, RANGE * HID)],
                    agg_hbm.at[pl.ds(pl.multiple_of(lo * HID, 8), RANGE * HID)])


def _silu(x):
    return x * (1.0 / (1.0 + jnp.exp(-x)))


def _mlp(h, w1, b1, w2, b2):
    t = jnp.maximum(jnp.dot(h, w1, preferred_element_type=jnp.float32) + b1, 0.0)
    return jnp.dot(t, w2, preferred_element_type=jnp.float32) + b2


def _embed_body(x_ref, wemb_ref, bemb_ref, w1_ref, b1_ref, w2_ref, b2_ref,
                h0_ref, m1_ref):
    h0 = _silu(x_ref[...] * wemb_ref[...] + bemb_ref[...])
    h0_ref[...] = h0
    m1_ref[...] = _mlp(h0, w1_ref[...], b1_ref[...], w2_ref[...], b2_ref[...])


def _combine_body(h_ref, agg_ref, w1_ref, b1_ref, w2_ref, b2_ref,
                  h1_ref, m2_ref):
    h1 = h_ref[...] + _silu(agg_ref[...])
    h1_ref[...] = h1
    m2_ref[...] = _mlp(h1, w1_ref[...], b1_ref[...], w2_ref[...], b2_ref[...])


def _final_body(h_ref, agg_ref, batch_ref, out_ref):
    h2 = h_ref[...] + _silu(agg_ref[...])
    gid = lax.broadcasted_iota(jnp.int32, (N_GRAPHS, N_NODES), 0)
    oh = (gid == batch_ref[...]).astype(jnp.float32)
    sums = jnp.dot(oh, h2, preferred_element_type=jnp.float32)
    counts = jnp.sum(oh, axis=1, keepdims=True)
    out_ref[...] = sums / jnp.maximum(counts, 1.0)


_F32 = jnp.float32

_embed = pl.pallas_call(
    _embed_body,
    out_shape=[jax.ShapeDtypeStruct((N_NODES, HID), _F32),
               jax.ShapeDtypeStruct((N_NODES, HID), _F32)],
)

_combine = pl.pallas_call(
    _combine_body,
    out_shape=[jax.ShapeDtypeStruct((N_NODES, HID), _F32),
               jax.ShapeDtypeStruct((N_NODES, HID), _F32)],
)

_final = pl.pallas_call(
    _final_body,
    out_shape=jax.ShapeDtypeStruct((N_GRAPHS, HID), _F32),
)


@functools.lru_cache(maxsize=1)
def _sc_kernels():
    mesh = plsc.VectorSubcoreMesh(core_axis_name="c", subcore_axis_name="s")
    partition = pl.kernel(
        _partition_body,
        out_type=[
            jax.ShapeDtypeStruct((NW * LISTCAP,), jnp.int32),
            jax.ShapeDtypeStruct((NW * 16,), jnp.int32),
        ],
        mesh=mesh,
        compiler_params=_SC_PARAMS,
        scratch_types=[
            pltpu.VMEM((CH_A,), jnp.int32),
            pltpu.VMEM((CH_A,), jnp.int32),
            pltpu.VMEM((BUFA,), jnp.int32),
            pltpu.VMEM((16,), jnp.int32),
        ],
    )
    segmax = pl.kernel(
        _segmax_body,
        out_type=jax.ShapeDtypeStruct((N_PAD * HID,), jnp.float32),
        mesh=mesh,
        compiler_params=_SC_PARAMS,
        scratch_types=[
            pltpu.VMEM(((RANGE + 1) * HID,), jnp.float32),
            pltpu.VMEM(((RANGE + 1) * HID,), jnp.float32),
            pltpu.VMEM((CH_B,), jnp.int32),
            pltpu.VMEM((CH_B,), jnp.int32),
            pltpu.VMEM((CH_B,), jnp.int32),
            pltpu.VMEM((CH_B,), jnp.int32),
            pltpu.VMEM((CH_B, HID), jnp.float32),
            pltpu.VMEM((CH_B, HID), jnp.float32),
            pltpu.VMEM((16,), jnp.int32),
            pltpu.SemaphoreType.DMA,
            pltpu.SemaphoreType.DMA,
            pltpu.SemaphoreType.DMA,
            pltpu.SemaphoreType.DMA,
        ],
    )
    return partition, segmax


def kernel(x, edge_index, batch, W_emb, b_emb,
           c1_W1, c1_b1, c1_W2, c1_b2,
           c2_W1, c2_b1, c2_W2, c2_b2):
    src = edge_index[0].astype(jnp.int32)
    dst = edge_index[1].astype(jnp.int32)
    batch2 = batch.astype(jnp.int32).reshape(1, N_NODES)
    bemb = b_emb.reshape(1, HID)
    c1b1 = c1_b1.reshape(1, HID)
    c1b2 = c1_b2.reshape(1, HID)
    c2b1 = c2_b1.reshape(1, HID)
    c2b2 = c2_b2.reshape(1, HID)

    _partition, _segmax = _sc_kernels()
    lists, counts = _partition(src, dst)
    h0, m1 = _embed(x, W_emb, bemb, c1_W1, c1b1, c1_W2, c1b2)
    agg1 = _segmax(m1, lists, counts).reshape(N_PAD, HID)[:N_NODES]
    h1, m2 = _combine(h0, agg1, c2_W1, c2b1, c2_W2, c2b2)
    agg2 = _segmax(m2, lists, counts).reshape(N_PAD, HID)[:N_NODES]
    return _final(h1, agg2, batch2)


# speedup vs baseline: 1.6095x; 1.1557x over previous
import functools

import jax
import jax.numpy as jnp
from jax import lax
from jax.experimental import pallas as pl
from jax.experimental.pallas import tpu as pltpu
from jax.experimental.pallas import tpu_sc as plsc

N_NODES = 10000
N_EDGES = 320000
HID = 128
N_GRAPHS = 16

NC = 2
NS = 16
NW = NC * NS
RANGE = 313
N_PAD = NW * RANGE
SHIFT = 1 << 14
PADVAL = RANGE * SHIFT

CH_A = 2000
NCH_A = N_EDGES // CH_A
SEG = 4096
BUF2 = 16 * SEG
TH = SEG - 126
OVFCAP = 82 * BUF2
MRGCAP = 67584
LISTCAP = OVFCAP + MRGCAP
BUFA = BUF2 + 2304
CH_B = 128
PAD_B = 2 * CH_B

_SC_PARAMS = pltpu.CompilerParams(needs_layout_passes=False)


def _wid():
    return lax.axis_index("s") * NC + lax.axis_index("c")


def _partition_body(src_hbm, dst_hbm, lists_hbm, counts_hbm,
                    sv0, sv1, dv0, dv1, outbuf, cnt_stage,
                    ss0, ss1, sd0, sd1):
    SV = (sv0, sv1)
    DV = (dv0, dv1)
    SS = (ss0, ss1)
    SD = (sd0, sd1)
    wid = _wid()
    lo = wid * RANGE
    hi = lo + RANGE
    lbase = wid * LISTCAP
    lanes = lax.iota(jnp.int32, 16)
    lane_base = lanes * SEG
    padv = jnp.full((16,), PADVAL, jnp.int32)

    def prefill(i, _):
        outbuf[pl.ds(i * 16, 16)] = padv
        return 0

    lax.fori_loop(0, BUF2 // 16, prefill, 0)

    def make_group(b):
        def group(j, lc):
            s = SV[b][pl.ds(j * 16, 16)]
            t = DV[b][pl.ds(j * 16, 16)]
            msk = (t >= lo) & (t < hi)
            packed = s + (t - lo) * SHIFT
            plsc.store_scatter(outbuf, [lane_base + lc], packed, mask=msk)
            return lc + msk.astype(jnp.int32)
        return group

    def stage_load(c, b):
        off = pl.multiple_of(c * CH_A, 8)
        pltpu.async_copy(src_hbm.at[pl.ds(off, CH_A)], SV[b], SS[b])
        pltpu.async_copy(dst_hbm.at[pl.ds(off, CH_A)], DV[b], SD[b])

    def stage_wait(b):
        pltpu.make_async_copy(src_hbm.at[pl.ds(0, CH_A)], SV[b], SS[b]).wait()
        pltpu.make_async_copy(dst_hbm.at[pl.ds(0, CH_A)], DV[b], SD[b]).wait()

    def do_flush(lc, ovf_w):
        pltpu.sync_copy(outbuf.at[pl.ds(0, BUF2)],
                        lists_hbm.at[pl.ds(pl.multiple_of(lbase + ovf_w, 128), BUF2)])
        lax.fori_loop(0, BUF2 // 16, prefill, 0)
        return jnp.zeros((16,), jnp.int32), ovf_w + BUF2

    def no_flush(lc, ovf_w):
        return lc, ovf_w

    stage_load(0, 0)

    def pair(t, carry):
        lc, ovf_w = carry
        for k in range(2):
            c = 2 * t + k
            b = k
            bn = 1 - k
            stage_wait(b)
            if k == 0:
                stage_load(c + 1, bn)
            else:
                @pl.when(c + 1 < NCH_A)
                def _():
                    stage_load(c + 1, bn)
            lc = lax.fori_loop(0, CH_A // 16, make_group(b), lc)
            big = (lc >= TH).astype(jnp.int32)
            anyb = big[0]
            for q in range(1, 16):
                anyb = anyb | big[q]
            lc, ovf_w = lax.cond(anyb > 0, do_flush, no_flush, lc, ovf_w)
        return lc, ovf_w

    lc, ovf_w = lax.fori_loop(0, NCH_A // 2, pair,
                              (jnp.zeros((16,), jnp.int32), jnp.int32(0)))

    tot = jnp.int32(0)
    for L in range(16):
        cl = lc[L]
        src_base = L * SEG
        dst_base = tot

        def cp(v, _):
            vec = outbuf[pl.ds(src_base + v * 16, 16)]
            outbuf[pl.ds(dst_base + v * 16, 16)] = vec
            return 0

        lax.fori_loop(0, (cl + 15) >> 4, cp, 0)
        tot = tot + cl

    for j in range(PAD_B // 16):
        outbuf[pl.ds(tot + j * 16, 16)] = padv
    cnt_p = ((tot + PAD_B - 1) >> 8) << 8

    def blk(b, _):
        pltpu.sync_copy(
            outbuf.at[pl.ds(b * 2048, 2048)],
            lists_hbm.at[pl.ds(pl.multiple_of(lbase + OVFCAP + b * 2048, 2048), 2048)])
        return 0

    lax.fori_loop(0, (cnt_p + 2047) >> 11, blk, 0)
    cnt_stage[...] = jnp.where(lanes == 0, ovf_w,
                               jnp.where(lanes == 1, cnt_p, 0))
    pltpu.sync_copy(cnt_stage, counts_hbm.at[pl.ds(pl.multiple_of(wid * 16, 16), 16)])


def _segmax_body(m_hbm, lists_hbm, counts_hbm, agg_hbm,
                 acc, acc2, p0, p1, s0, s1, r0, r1, cnt_stage,
                 g0, g1, l0, l1):
    P = (p0, p1)
    S = (s0, s1)
    RW = (r0, r1)
    G = (g0, g1)
    L = (l0, l1)
    wid = _wid()
    lo = wid * RANGE
    lbase = wid * LISTCAP

    pltpu.sync_copy(counts_hbm.at[pl.ds(pl.multiple_of(wid * 16, 16), 16)], cnt_stage)
    cv = cnt_stage[pl.ds(0, 16)]
    novf = cv[0] >> 7
    nch = novf + (cv[1] >> 7)

    ninf = jnp.full((16,), -jnp.inf, jnp.float32)

    def init(i, _):
        acc[pl.ds(i * 16, 16)] = ninf
        acc2[pl.ds(i * 16, 16)] = ninf
        return 0

    lax.fori_loop(0, (RANGE + 1) * HID // 16, init, 0)

    def chunk_addr(c):
        return lbase + jnp.where(c < novf, c * CH_B,
                                 OVFCAP + (c - novf) * CH_B)

    def list_load(c, b):
        pltpu.async_copy(
            lists_hbm.at[pl.ds(pl.multiple_of(chunk_addr(c), CH_B), CH_B)],
            P[b].at[pl.ds(0, CH_B)], L[b])

    def list_wait(b):
        pltpu.make_async_copy(
            lists_hbm.at[pl.ds(pl.multiple_of(lbase, CH_B), CH_B)],
            P[b].at[pl.ds(0, CH_B)], L[b]).wait()

    def unpack_and_gather(b):
        for j in range(CH_B // 16):
            pv = P[b][pl.ds(j * 16, 16)]
            S[b][pl.ds(j * 16, 16)] = pv & (SHIFT - 1)
        pltpu.async_copy(m_hbm.at[S[b]], RW[b], G[b])

    def gather_wait(b):
        pltpu.make_async_copy(m_hbm.at[S[b]], RW[b], G[b]).wait()

    def accumulate(b):
        rows = RW[b]
        pk_buf = P[b]

        def grp(g, _):
            pkv = pk_buf[pl.ds(pl.multiple_of(g * 16, 16), 16)]
            dls = (pkv >> 14) * HID
            for k in range(16):
                base = dls[k]
                e = g * 16 + k
                a = acc if (k & 1) == 0 else acc2
                for c in range(HID // 16):
                    off = base + c * 16
                    a[pl.ds(off, 16)] = jnp.maximum(
                        a[pl.ds(off, 16)], rows[e, pl.ds(c * 16, 16)])
            return 0

        lax.fori_loop(0, CH_B // 16, grp, 0)

    @pl.when(nch > 0)
    def _():
        list_load(0, 0)
        list_load(1, 1)
        list_wait(0)
        unpack_and_gather(0)

    def pair(t, _):
        for k in range(2):
            c = t * 2 + k
            b = k
            bn = (k + 1) % 2
            gather_wait(b)

            @pl.when(c + 1 < nch)
            def _():
                list_wait(bn)
                unpack_and_gather(bn)

            accumulate(b)

            @pl.when(c + 2 < nch)
            def _():
                list_load(c + 2, b)
        return 0

    lax.fori_loop(0, nch // 2, pair, 0)

    def fix(i, _):
        v = jnp.maximum(acc[pl.ds(i * 16, 16)], acc2[pl.ds(i * 16, 16)])
        acc[pl.ds(i * 16, 16)] = jnp.where(v == -jnp.inf, 0.0, v)
        return 0

    lax.fori_loop(0, RANGE * HID // 16, fix, 0)
    pltpu.sync_copy(acc.at[pl.ds(0, RANGE * HID)],
                    agg_hbm.at[pl.ds(pl.multiple_of(lo * HID, 8), RANGE * HID)])


def _silu(x):
    return x * (1.0 / (1.0 + jnp.exp(-x)))


def _mlp(h, w1, b1, w2, b2):
    t = jnp.maximum(jnp.dot(h, w1, preferred_element_type=jnp.float32) + b1, 0.0)
    return jnp.dot(t, w2, preferred_element_type=jnp.float32) + b2


def _embed_body(x_ref, wemb_ref, bemb_ref, w1_ref, b1_ref, w2_ref, b2_ref,
                h0_ref, m1_ref):
    h0 = _silu(x_ref[...] * wemb_ref[...] + bemb_ref[...])
    h0_ref[...] = h0
    m1_ref[...] = _mlp(h0, w1_ref[...], b1_ref[...], w2_ref[...], b2_ref[...])


def _combine_body(h_ref, agg_ref, w1_ref, b1_ref, w2_ref, b2_ref,
                  h1_ref, m2_ref):
    h1 = h_ref[...] + _silu(agg_ref[...])
    h1_ref[...] = h1
    m2_ref[...] = _mlp(h1, w1_ref[...], b1_ref[...], w2_ref[...], b2_ref[...])


def _final_body(h_ref, agg_ref, batch_ref, out_ref):
    h2 = h_ref[...] + _silu(agg_ref[...])
    gid = lax.broadcasted_iota(jnp.int32, (N_GRAPHS, N_NODES), 0)
    oh = (gid == batch_ref[...]).astype(jnp.float32)
    sums = jnp.dot(oh, h2, preferred_element_type=jnp.float32)
    counts = jnp.sum(oh, axis=1, keepdims=True)
    out_ref[...] = sums / jnp.maximum(counts, 1.0)


_F32 = jnp.float32

_embed = pl.pallas_call(
    _embed_body,
    out_shape=[jax.ShapeDtypeStruct((N_NODES, HID), _F32),
               jax.ShapeDtypeStruct((N_NODES, HID), _F32)],
)

_combine = pl.pallas_call(
    _combine_body,
    out_shape=[jax.ShapeDtypeStruct((N_NODES, HID), _F32),
               jax.ShapeDtypeStruct((N_NODES, HID), _F32)],
)

_final = pl.pallas_call(
    _final_body,
    out_shape=jax.ShapeDtypeStruct((N_GRAPHS, HID), _F32),
)


@functools.lru_cache(maxsize=1)
def _sc_kernels():
    mesh = plsc.VectorSubcoreMesh(core_axis_name="c", subcore_axis_name="s")
    partition = pl.kernel(
        _partition_body,
        out_type=[
            jax.ShapeDtypeStruct((NW * LISTCAP,), jnp.int32),
            jax.ShapeDtypeStruct((NW * 16,), jnp.int32),
        ],
        mesh=mesh,
        compiler_params=_SC_PARAMS,
        scratch_types=[
            pltpu.VMEM((CH_A,), jnp.int32),
            pltpu.VMEM((CH_A,), jnp.int32),
            pltpu.VMEM((CH_A,), jnp.int32),
            pltpu.VMEM((CH_A,), jnp.int32),
            pltpu.VMEM((BUFA,), jnp.int32),
            pltpu.VMEM((16,), jnp.int32),
            pltpu.SemaphoreType.DMA,
            pltpu.SemaphoreType.DMA,
            pltpu.SemaphoreType.DMA,
            pltpu.SemaphoreType.DMA,
        ],
    )
    segmax = pl.kernel(
        _segmax_body,
        out_type=jax.ShapeDtypeStruct((N_PAD * HID,), jnp.float32),
        mesh=mesh,
        compiler_params=_SC_PARAMS,
        scratch_types=[
            pltpu.VMEM(((RANGE + 1) * HID,), jnp.float32),
            pltpu.VMEM(((RANGE + 1) * HID,), jnp.float32),
            pltpu.VMEM((CH_B,), jnp.int32),
            pltpu.VMEM((CH_B,), jnp.int32),
            pltpu.VMEM((CH_B,), jnp.int32),
            pltpu.VMEM((CH_B,), jnp.int32),
            pltpu.VMEM((CH_B, HID), jnp.float32),
            pltpu.VMEM((CH_B, HID), jnp.float32),
            pltpu.VMEM((16,), jnp.int32),
            pltpu.SemaphoreType.DMA,
            pltpu.SemaphoreType.DMA,
            pltpu.SemaphoreType.DMA,
            pltpu.SemaphoreType.DMA,
        ],
    )
    return partition, segmax


def kernel(x, edge_index, batch, W_emb, b_emb,
           c1_W1, c1_b1, c1_W2, c1_b2,
           c2_W1, c2_b1, c2_W2, c2_b2):
    src = edge_index[0].astype(jnp.int32)
    dst = edge_index[1].astype(jnp.int32)
    batch2 = batch.astype(jnp.int32).reshape(1, N_NODES)
    bemb = b_emb.reshape(1, HID)
    c1b1 = c1_b1.reshape(1, HID)
    c1b2 = c1_b2.reshape(1, HID)
    c2b1 = c2_b1.reshape(1, HID)
    c2b2 = c2_b2.reshape(1, HID)

    _partition, _segmax = _sc_kernels()
    lists, counts = _partition(src, dst)
    h0, m1 = _embed(x, W_emb, bemb, c1_W1, c1b1, c1_W2, c1b2)
    agg1 = _segmax(m1, lists, counts).reshape(N_PAD, HID)[:N_NODES]
    h1, m2 = _combine(h0, agg1, c2_W1, c2b1, c2_W2, c2b2)
    agg2 = _segmax(m2, lists, counts).reshape(N_PAD, HID)[:N_NODES]
    return _final(h1, agg2, batch2)
